# Initial kernel scaffold; baseline (speedup 1.0000x reference)
import jax
import jax.numpy as jnp
from jax.experimental import pallas as pl


def kernel(x, edge_index, edge_attr, batch, node_W, edge_W, in_g, in_b, Wq, Wk, Wv, We, Wo, Woe, ln1_g, ln1_b, ln1e_g, ln1e_b, fW1, fb1, fW2, fb2, feW1, feb1, feW2, feb2, ln2_g, ln2_b, ln2e_g, ln2e_b, ro_g, ro_b, muW1, mub1, muW2, mub2, lvW1, lvb1, lvW2, lvb2):
    raise NotImplementedError("write your pallas kernel here")


# speedup vs baseline: 12.8902x; 12.8902x over previous
import functools

import jax
import jax.numpy as jnp
from jax import lax
from jax.experimental import pallas as pl
from jax.experimental.pallas import tpu as pltpu
from jax.experimental.pallas import tpu_sc as plsc

N = 10000
E = 160000
D = 128
DE = 16
H = 8
DH = 16
L = 4
NG = 64
FF = 2 * D

NC = 2
NS = 16
NW = NC * NS
CH = 128
NCHUNK = 41
EPW = CH * NCHUNK
EPAD = EPW * NW
NTPT = 320
NACC = NW * NTPT

_INTERPRET = False


def _ln(x, g, b):
    m = jnp.mean(x, axis=-1, keepdims=True)
    v = jnp.mean((x - m) * (x - m), axis=-1, keepdims=True)
    return (x - m) * jax.lax.rsqrt(v + 1e-5) * g + b


def _gelu(x):
    c = 0.7978845608028654
    return 0.5 * x * (1.0 + jnp.tanh(c * (x + 0.044715 * x * x * x)))


def _embed_node_body(x_ref, w_ref, g_ref, b_ref, wq_ref, wkv_ref,
                     h_ref, q_ref, kv_ref):
    h = _ln(jnp.dot(x_ref[...], w_ref[...], preferred_element_type=jnp.float32),
            g_ref[...], b_ref[...])
    h_ref[...] = h
    q_ref[...] = jnp.dot(h, wq_ref[...], preferred_element_type=jnp.float32)
    kv_ref[...] = jnp.dot(h, wkv_ref[...], preferred_element_type=jnp.float32)


def _embed_node(x, node_W, in_g, in_b, Wq0, Wkv0):
    BN = 1000
    grid = (N // BN,)
    return pl.pallas_call(
        _embed_node_body,
        grid=grid,
        in_specs=[
            pl.BlockSpec((BN, D), lambda i: (i, 0)),
            pl.BlockSpec((D, D), lambda i: (0, 0)),
            pl.BlockSpec((1, D), lambda i: (0, 0)),
            pl.BlockSpec((1, D), lambda i: (0, 0)),
            pl.BlockSpec((D, D), lambda i: (0, 0)),
            pl.BlockSpec((D, 2 * D), lambda i: (0, 0)),
        ],
        out_specs=[
            pl.BlockSpec((BN, D), lambda i: (i, 0)),
            pl.BlockSpec((BN, D), lambda i: (i, 0)),
            pl.BlockSpec((BN, 2 * D), lambda i: (i, 0)),
        ],
        out_shape=[
            jax.ShapeDtypeStruct((N, D), jnp.float32),
            jax.ShapeDtypeStruct((N, D), jnp.float32),
            jax.ShapeDtypeStruct((N, 2 * D), jnp.float32),
        ],
        interpret=_INTERPRET,
    )(x, node_W, in_g, in_b, Wq0, Wkv0)


def _embed_edge_body(ea_ref, w_ref, e_ref):
    e_ref[...] = jnp.dot(ea_ref[...], w_ref[...],
                         preferred_element_type=jnp.float32)


def _embed_edge(ea_pad, edge_W):
    BE = 2048
    grid = (EPAD // BE,)
    return pl.pallas_call(
        _embed_edge_body,
        grid=grid,
        in_specs=[
            pl.BlockSpec((BE, DE), lambda i: (i, 0)),
            pl.BlockSpec((DE, D), lambda i: (0, 0)),
        ],
        out_specs=pl.BlockSpec((BE, D), lambda i: (i, 0)),
        out_shape=jax.ShapeDtypeStruct((EPAD, D), jnp.float32),
        interpret=_INTERPRET,
    )(ea_pad, edge_W)


def _edge_dense_body(e_ref, qd_ref, kvs_ref, we_ref, woe_ref, a_ref, msel_ref,
                     l1g_ref, l1b_ref, f1_ref, fb1_ref, f2_ref, fb2_ref,
                     l2g_ref, l2b_ref,
                     eo_ref, num_ref, den_ref):
    e = e_ref[...]
    ep = jnp.dot(e, we_ref[...], preferred_element_type=jnp.float32)
    ks = kvs_ref[:, :D]
    vs = kvs_ref[:, D:]
    score = qd_ref[...] * ks * ep * 0.25
    e_new = jnp.dot(score, woe_ref[...], preferred_element_type=jnp.float32)
    t = jnp.dot(score, a_ref[...], preferred_element_type=jnp.float32)
    sbc = jnp.exp(jnp.clip(t, -5.0, 5.0))
    num_ref[...] = sbc * vs
    den_ref[...] = jnp.dot(sbc, msel_ref[...], preferred_element_type=jnp.float32)
    e1 = _ln(e + e_new, l1g_ref[...], l1b_ref[...])
    ff = jnp.dot(_gelu(jnp.dot(e1, f1_ref[...], preferred_element_type=jnp.float32)
                       + fb1_ref[...]),
                 f2_ref[...], preferred_element_type=jnp.float32) + fb2_ref[...]
    eo_ref[...] = _ln(e1 + ff, l2g_ref[...], l2b_ref[...])


def _edge_dense(e, qd, kvs, We_l, Woe_l, A, Msel, l1g, l1b, f1, fb1, f2, fb2,
                l2g, l2b):
    BE = 512
    grid = (EPAD // BE,)
    cmap = lambda i: (0, 0)
    return pl.pallas_call(
        _edge_dense_body,
        grid=grid,
        in_specs=[
            pl.BlockSpec((BE, D), lambda i: (i, 0)),
            pl.BlockSpec((BE, D), lambda i: (i, 0)),
            pl.BlockSpec((BE, 2 * D), lambda i: (i, 0)),
            pl.BlockSpec((D, D), cmap),
            pl.BlockSpec((D, D), cmap),
            pl.BlockSpec((D, D), cmap),
            pl.BlockSpec((D, DH), cmap),
            pl.BlockSpec((1, D), cmap),
            pl.BlockSpec((1, D), cmap),
            pl.BlockSpec((D, FF), cmap),
            pl.BlockSpec((1, FF), cmap),
            pl.BlockSpec((FF, D), cmap),
            pl.BlockSpec((1, D), cmap),
            pl.BlockSpec((1, D), cmap),
            pl.BlockSpec((1, D), cmap),
        ],
        out_specs=[
            pl.BlockSpec((BE, D), lambda i: (i, 0)),
            pl.BlockSpec((BE, D), lambda i: (i, 0)),
            pl.BlockSpec((BE, DH), lambda i: (i, 0)),
        ],
        out_shape=[
            jax.ShapeDtypeStruct((EPAD, D), jnp.float32),
            jax.ShapeDtypeStruct((EPAD, D), jnp.float32),
            jax.ShapeDtypeStruct((EPAD, DH), jnp.float32),
        ],
        interpret=_INTERPRET,
    )(e, qd, kvs, We_l, Woe_l, A, Msel, l1g, l1b, f1, fb1, f2, fb2, l2g, l2b)


def _node_dense_body(h_ref, num_ref, den_ref, mexp_ref, wo_ref,
                     l1g_ref, l1b_ref, f1_ref, fb1_ref, f2_ref, fb2_ref,
                     l2g_ref, l2b_ref, wq_ref, wkv_ref,
                     ho_ref, q_ref, kv_ref):
    num = num_ref[...]
    den = den_ref[...]
    denbc = jnp.dot(den, mexp_ref[...], preferred_element_type=jnp.float32) + 1e-6
    h_attn = num / denbc
    h_new = jnp.dot(h_attn, wo_ref[...], preferred_element_type=jnp.float32)
    h1 = _ln(h_ref[...] + h_new, l1g_ref[...], l1b_ref[...])
    ff = jnp.dot(_gelu(jnp.dot(h1, f1_ref[...], preferred_element_type=jnp.float32)
                       + fb1_ref[...]),
                 f2_ref[...], preferred_element_type=jnp.float32) + fb2_ref[...]
    h2 = _ln(h1 + ff, l2g_ref[...], l2b_ref[...])
    ho_ref[...] = h2
    q_ref[...] = jnp.dot(h2, wq_ref[...], preferred_element_type=jnp.float32)
    kv_ref[...] = jnp.dot(h2, wkv_ref[...], preferred_element_type=jnp.float32)


def _node_dense(h, num, den, Mexp, Wo_l, l1g, l1b, f1, fb1, f2,
                fb2, l2g, l2b, Wq_n, Wkv_n):
    BN = 1000
    grid = (N // BN,)
    cmap = lambda i: (0, 0)
    return pl.pallas_call(
        _node_dense_body,
        grid=grid,
        in_specs=[
            pl.BlockSpec((BN, D), lambda i: (i, 0)),
            pl.BlockSpec((BN, D), lambda i: (i, 0)),
            pl.BlockSpec((BN, DH), lambda i: (i, 0)),
            pl.BlockSpec((DH, D), cmap),
            pl.BlockSpec((D, D), cmap),
            pl.BlockSpec((1, D), cmap),
            pl.BlockSpec((1, D), cmap),
            pl.BlockSpec((D, FF), cmap),
            pl.BlockSpec((1, FF), cmap),
            pl.BlockSpec((FF, D), cmap),
            pl.BlockSpec((1, D), cmap),
            pl.BlockSpec((1, D), cmap),
            pl.BlockSpec((1, D), cmap),
            pl.BlockSpec((D, D), cmap),
            pl.BlockSpec((D, 2 * D), cmap),
        ],
        out_specs=[
            pl.BlockSpec((BN, D), lambda i: (i, 0)),
            pl.BlockSpec((BN, D), lambda i: (i, 0)),
            pl.BlockSpec((BN, 2 * D), lambda i: (i, 0)),
        ],
        out_shape=[
            jax.ShapeDtypeStruct((N, D), jnp.float32),
            jax.ShapeDtypeStruct((N, D), jnp.float32),
            jax.ShapeDtypeStruct((N, 2 * D), jnp.float32),
        ],
        interpret=_INTERPRET,
    )(h, num, den, Mexp, Wo_l, l1g, l1b, f1, fb1, f2, fb2, l2g,
      l2b, Wq_n, Wkv_n)


def _pool_body(h_ref, b_ref, g_ref):
    @pl.when(pl.program_id(0) == 0)
    def _():
        g_ref[...] = jnp.zeros_like(g_ref)

    b = b_ref[0]
    oh = (lax.broadcasted_iota(jnp.int32, (NG,) + b.shape[1:], 0) == b
          ).astype(jnp.float32)
    g_ref[...] += jnp.dot(oh, h_ref[...], preferred_element_type=jnp.float32)


def _pool(h, batch_r):
    BN = 1000
    grid = (N // BN,)
    return pl.pallas_call(
        _pool_body,
        grid=grid,
        in_specs=[
            pl.BlockSpec((BN, D), lambda i: (i, 0)),
            pl.BlockSpec((1, 1, BN), lambda i: (i, 0, 0)),
        ],
        out_specs=pl.BlockSpec((NG, D), lambda i: (0, 0)),
        out_shape=jax.ShapeDtypeStruct((NG, D), jnp.float32),
        interpret=_INTERPRET,
    )(h, batch_r)


def _head_body(g_ref, rg_ref, rb_ref, mw1_ref, mb1_ref, mw2_ref, mb2_ref,
               lw1_ref, lb1_ref, lw2_ref, lb2_ref, mu_ref, lv_ref):
    gn = _ln(g_ref[...], rg_ref[...], rb_ref[...])
    mu = jnp.dot(_gelu(jnp.dot(gn, mw1_ref[...], preferred_element_type=jnp.float32)
                       + mb1_ref[...]),
                 mw2_ref[...], preferred_element_type=jnp.float32) + mb2_ref[...]
    lv = jnp.dot(_gelu(jnp.dot(gn, lw1_ref[...], preferred_element_type=jnp.float32)
                       + lb1_ref[...]),
                 lw2_ref[...], preferred_element_type=jnp.float32) + lb2_ref[...]
    mu_ref[...] = mu
    lv_ref[...] = jnp.clip(lv, -10.0, 10.0)


def _head(g, ro_g, ro_b, muW1, mub1, muW2p, mub2p, lvW1, lvb1, lvW2p, lvb2p):
    cmap = lambda: (0, 0)
    return pl.pallas_call(
        _head_body,
        grid=(),
        in_specs=[pl.BlockSpec((NG, D), cmap)] +
                 [pl.BlockSpec((1, D), cmap)] * 2 +
                 [pl.BlockSpec((D, D), cmap), pl.BlockSpec((1, D), cmap),
                  pl.BlockSpec((D, D), cmap), pl.BlockSpec((1, D), cmap)] * 2,
        out_specs=[pl.BlockSpec((NG, D), cmap), pl.BlockSpec((NG, D), cmap)],
        out_shape=[jax.ShapeDtypeStruct((NG, D), jnp.float32),
                   jax.ShapeDtypeStruct((NG, D), jnp.float32)],
        interpret=_INTERPRET,
    )(g, ro_g, ro_b, muW1, mub1, muW2p, mub2p, lvW1, lvb1, lvW2p, lvb2p)


def _sc_permute_body(ea_h, ord_h, eas_o, idxo, obuf, sem1):
    wid = lax.axis_index("c") * NS + lax.axis_index("s")
    pltpu.sync_copy(ord_h.at[wid], idxo)

    def step(j, carry):
        base = wid * EPW + j * CH
        pltpu.async_copy(ea_h.at[idxo.at[j]], obuf, sem1).wait()
        pltpu.sync_copy(obuf, eas_o.at[pl.ds(base, CH)])
        return carry

    lax.fori_loop(0, NCHUNK, step, 0)


def _sc_permute(e0, ord_r):
    mesh = plsc.VectorSubcoreMesh(core_axis_name="c", subcore_axis_name="s")
    return pl.kernel(
        _sc_permute_body,
        out_type=jax.ShapeDtypeStruct((EPAD, D), jnp.float32),
        mesh=mesh,
        scratch_types=[
            pltpu.VMEM((NCHUNK, CH), jnp.int32),
            pltpu.VMEM((CH, D), jnp.float32),
            pltpu.SemaphoreType.DMA,
        ],
    )(e0, ord_r)


def _sc_gather_body(q_h, kv_h, dst_h, src_h, qd_o, kvs_o,
                    idxd, idxs, qbuf, kvbuf, sem1, sem2):
    wid = lax.axis_index("c") * NS + lax.axis_index("s")
    pltpu.sync_copy(dst_h.at[wid], idxd)
    pltpu.sync_copy(src_h.at[wid], idxs)

    def step(j, carry):
        base = wid * EPW + j * CH
        cp1 = pltpu.async_copy(q_h.at[idxd.at[j]], qbuf, sem1)
        cp2 = pltpu.async_copy(kv_h.at[idxs.at[j]], kvbuf, sem2)
        cp1.wait()
        cp2.wait()
        pltpu.sync_copy(qbuf, qd_o.at[pl.ds(base, CH)])
        pltpu.sync_copy(kvbuf, kvs_o.at[pl.ds(base, CH)])
        return carry

    lax.fori_loop(0, NCHUNK, step, 0)


def _sc_gather(q, kv, dst_r, src_r):
    mesh = plsc.VectorSubcoreMesh(core_axis_name="c", subcore_axis_name="s")
    return pl.kernel(
        _sc_gather_body,
        out_type=(
            jax.ShapeDtypeStruct((EPAD, D), jnp.float32),
            jax.ShapeDtypeStruct((EPAD, 2 * D), jnp.float32),
        ),
        mesh=mesh,
        scratch_types=[
            pltpu.VMEM((NCHUNK, CH), jnp.int32),
            pltpu.VMEM((NCHUNK, CH), jnp.int32),
            pltpu.VMEM((CH, D), jnp.float32),
            pltpu.VMEM((CH, 2 * D), jnp.float32),
            pltpu.SemaphoreType.DMA,
            pltpu.SemaphoreType.DMA,
        ],
    )(q, kv, dst_r, src_r)


def _sc_segsum_body(num_h, den_h, dstv_h, tb_h, outn_o, outd_o,
                    tbv, dstbuf, nbuf, dbuf, accn, accd):
    wid = lax.axis_index("c") * NS + lax.axis_index("s")
    n0 = wid * NTPT
    pltpu.sync_copy(tb_h.at[wid], tbv)
    zv = jnp.zeros((16,), jnp.float32)

    def zrow(i, c):
        def zcol(k, c2):
            accn[i, pl.ds(k * 16, 16)] = zv
            return c2
        lax.fori_loop(0, D // 16, zcol, c)
        accd[i, pl.ds(0, 16)] = zv
        return c

    lax.fori_loop(0, NTPT + 1, zrow, 0)

    tvec = tbv[pl.ds(0, 16)]
    e0 = tvec[0]
    e1 = jnp.minimum(tvec[1], E)
    c0 = e0 // CH
    c1 = (e1 + CH - 1) // CH

    def chunk(c, carry):
        base = c * CH
        pltpu.sync_copy(num_h.at[pl.ds(base, CH)], nbuf)
        pltpu.sync_copy(den_h.at[pl.ds(base, CH)], dbuf)
        pltpu.sync_copy(dstv_h.at[pl.ds(base, CH)], dstbuf)

        def grp(k, c2):
            dvec = dstbuf[pl.ds(k * 16, 16)]
            for j in range(16):
                i = k * 16 + j
                g = base + i
                d = dvec[j]
                ok = (g >= e0) & (g < e1)
                row = jnp.where(ok, d - n0, NTPT)
                for kk in range(D // 16):
                    sl = pl.ds(kk * 16, 16)
                    accn[row, sl] = accn[row, sl] + nbuf[i, sl]
                sl = pl.ds(0, 16)
                accd[row, sl] = accd[row, sl] + dbuf[i, sl]
            return c2

        lax.fori_loop(0, CH // 16, grp, carry)
        return carry

    lax.fori_loop(c0, c1, chunk, 0)
    pltpu.sync_copy(accn.at[pl.ds(0, NTPT)], outn_o.at[pl.ds(n0, NTPT)])
    pltpu.sync_copy(accd.at[pl.ds(0, NTPT)], outd_o.at[pl.ds(n0, NTPT)])


def _sc_segsum(num_c, den_c, dstv, tb):
    mesh = plsc.VectorSubcoreMesh(core_axis_name="c", subcore_axis_name="s")
    return pl.kernel(
        _sc_segsum_body,
        out_type=(
            jax.ShapeDtypeStruct((NACC, D), jnp.float32),
            jax.ShapeDtypeStruct((NACC, DH), jnp.float32),
        ),
        mesh=mesh,
        scratch_types=[
            pltpu.VMEM((16,), jnp.int32),
            pltpu.VMEM((CH,), jnp.int32),
            pltpu.VMEM((CH, D), jnp.float32),
            pltpu.VMEM((CH, DH), jnp.float32),
            pltpu.VMEM((NTPT + 16, D), jnp.float32),
            pltpu.VMEM((NTPT + 16, DH), jnp.float32),
        ],
    )(num_c, den_c, dstv, tb)


def kernel(x, edge_index, edge_attr, batch, node_W, edge_W, in_g, in_b, Wq,
           Wk, Wv, We, Wo, Woe, ln1_g, ln1_b, ln1e_g, ln1e_b, fW1, fb1, fW2,
           fb2, feW1, feb1, feW2, feb2, ln2_g, ln2_b, ln2e_g, ln2e_b, ro_g,
           ro_b, muW1, mub1, muW2, mub2, lvW1, lvb1, lvW2, lvb2):
    f32 = jnp.float32
    src = edge_index[0].astype(jnp.int32)
    dst = edge_index[1].astype(jnp.int32)
    pad = EPAD - E
    src_p = jnp.concatenate([src, jnp.zeros((pad,), jnp.int32)])
    dst_p = jnp.concatenate([dst, jnp.full((pad,), N, jnp.int32)])
    order = jnp.argsort(dst_p)
    dst_s = dst_p[order]
    src_s = src_p[order]
    tb = jnp.searchsorted(dst_s, jnp.arange(NW + 1, dtype=jnp.int32) * NTPT
                          ).astype(jnp.int32)
    tb = jnp.concatenate(
        [tb[:NW, None], tb[1:NW + 1, None],
         jnp.zeros((NW, 14), jnp.int32)], axis=1)
    ord_r = order.astype(jnp.int32).reshape(NW, NCHUNK, CH)
    src_r = src_s.reshape(NW, NCHUNK, CH)
    dst_g = jnp.where(dst_s >= N, 0, dst_s)
    dst_r = dst_g.reshape(NW, NCHUNK, CH)
    ea_pad = jnp.concatenate([edge_attr, jnp.zeros((pad, DE), f32)])

    ii = jnp.arange(D, dtype=jnp.int32)
    A = (ii[:, None] // DH == ii[None, :] // DH).astype(f32)
    hh = jnp.arange(DH, dtype=jnp.int32)
    Msel = ((ii[:, None] % DH == 0) & (ii[:, None] // DH == hh[None, :])
            ).astype(f32)
    Mexp = (hh[:, None] == ii[None, :] // DH).astype(f32)

    r2 = lambda a: a.reshape(1, -1)
    Wkv = jnp.concatenate([Wk, Wv], axis=2)

    h, q, kv = _embed_node(x, node_W, r2(in_g), r2(in_b), Wq[0], Wkv[0])
    e0u = _embed_edge(ea_pad, edge_W)
    e = _sc_permute(e0u, ord_r)

    for l in range(L):
        qd, kvs = _sc_gather(q, kv, dst_r, src_r)
        e, num_c, den_c = _edge_dense(
            e, qd, kvs, We[l], Woe[l], A, Msel, r2(ln1e_g[l]), r2(ln1e_b[l]),
            feW1[l], r2(feb1[l]), feW2[l], r2(feb2[l]), r2(ln2e_g[l]),
            r2(ln2e_b[l]))
        num, den = _sc_segsum(num_c, den_c, dst_s, tb)
        nl = min(l + 1, L - 1)
        h, q, kv = _node_dense(
            h, num[:N], den[:N], Mexp,
            Wo[l], r2(ln1_g[l]), r2(ln1_b[l]), fW1[l], r2(fb1[l]), fW2[l],
            r2(fb2[l]), r2(ln2_g[l]), r2(ln2_b[l]), Wq[nl], Wkv[nl])

    batch_r = batch.astype(jnp.int32).reshape(N // 1000, 1, 1000)
    g = _pool(h, batch_r)

    zpad = jnp.zeros((D, D - 1), f32)
    muW2p = jnp.concatenate([muW2, zpad], axis=1)
    lvW2p = jnp.concatenate([lvW2, zpad], axis=1)
    bpad = jnp.zeros((1, D - 1), f32)
    mub2p = jnp.concatenate([mub2.reshape(1, 1), bpad], axis=1)
    lvb2p = jnp.concatenate([lvb2.reshape(1, 1), bpad], axis=1)
    mu128, lv128 = _head(g, r2(ro_g), r2(ro_b), muW1, r2(mub1), muW2p, mub2p,
                         lvW1, r2(lvb1), lvW2p, lvb2p)
    return (mu128[:, :1], lv128[:, :1])
